# baseline (speedup 1.0000x reference)
import functools

import jax
import jax.numpy as jnp
from jax import lax
from jax.experimental import pallas as pl
from jax.experimental.pallas import tpu as pltpu
from jax.experimental.pallas import tpu_sc as plsc

N_NODES = 10000
N_EDGES = 320000
D_SCALAR = 128
D_EDGE = 16
D_HIDDEN = 128

_A_BLK = 1000


def _tables_body(ns_ref, w_ref, out_ref):
    out_ref[...] = jnp.dot(ns_ref[...], w_ref[0],
                           preferred_element_type=jnp.float32)


def _node_tables(node_scalars, w1_nodes_stacked):
    return pl.pallas_call(
        _tables_body,
        grid=(2, N_NODES // _A_BLK),
        in_specs=[
            pl.BlockSpec((_A_BLK, D_SCALAR), lambda t, i: (i, 0)),
            pl.BlockSpec((1, D_SCALAR, D_HIDDEN), lambda t, i: (t, 0, 0)),
        ],
        out_specs=pl.BlockSpec((_A_BLK, D_HIDDEN),
                               lambda t, i: (t * (N_NODES // _A_BLK) + i, 0)),
        out_shape=jax.ShapeDtypeStruct((2 * N_NODES, D_HIDDEN), jnp.float32),
    )(node_scalars, w1_nodes_stacked)


_NW = 32
_K = 5
_E_CHUNK = N_EDGES // _K
_EP = _E_CHUNK // _NW
_C = 400
_NCHUNK = _EP // _C


_GATHER_ADD_CACHE = []


def _gather_add_build():
    if _GATHER_ADD_CACHE:
        return _GATHER_ADD_CACHE[0]
    mesh = plsc.VectorSubcoreMesh(core_axis_name="c", subcore_axis_name="s")

    @functools.partial(
        pl.kernel,
        out_type=jax.ShapeDtypeStruct((_E_CHUNK, D_HIDDEN), jnp.float32),
        mesh=mesh,
        scratch_types=[
            pltpu.VMEM((_EP,), jnp.int32),
            pltpu.VMEM((_EP,), jnp.int32),
            pltpu.VMEM((_C, D_HIDDEN), jnp.float32),
            pltpu.VMEM((_C, D_HIDDEN), jnp.float32),
            pltpu.SemaphoreType.DMA,
            pltpu.SemaphoreType.DMA,
            pltpu.SemaphoreType.DMA,
            pltpu.SemaphoreType.DMA,
        ],
    )
    def gather_add(table_hbm, src_hbm, dst_hbm, out_hbm,
                   idx_s, idx_d, buf0, buf1, gs0, gs1, ws0, ws1):
        wid = lax.axis_index("s") * 2 + lax.axis_index("c")
        base = pl.multiple_of(wid * _EP, 8)
        bufs = (buf0, buf1)
        gsems = (gs0, gs1)
        wsems = (ws0, ws1)

        pltpu.sync_copy(src_hbm.at[pl.ds(base, _EP)], idx_s)
        pltpu.sync_copy(dst_hbm.at[pl.ds(base, _EP)], idx_d)
        for i in range(_EP // 16):
            sl = pl.ds(i * 16, 16)
            idx_d[sl] = idx_d[sl] + N_NODES

        def g1(ci):
            s = ci % 2
            return pltpu.async_copy(
                table_hbm.at[idx_s.at[pl.ds(ci * _C, _C)]], bufs[s], gsems[s])

        def g2(ci):
            s = ci % 2
            return pltpu.async_copy(
                table_hbm.at[idx_d.at[pl.ds(ci * _C, _C)]], bufs[s], gsems[s],
                add=True)

        def wb(ci):
            s = ci % 2
            return pltpu.async_copy(
                bufs[s], out_hbm.at[pl.ds(base + ci * _C, _C)], wsems[s])

        wbd = [None] * _NCHUNK
        d = g1(0)
        for ci in range(_NCHUNK):
            d.wait()
            dg2 = g2(ci)
            if ci >= 1:
                wbd[ci - 1].wait()
            if ci + 1 < _NCHUNK:
                d = g1(ci + 1)
            dg2.wait()
            wbd[ci] = wb(ci)
        wbd[_NCHUNK - 1].wait()

    _GATHER_ADD_CACHE.append(gather_add)
    return gather_add


_E_BLK = 8000


def _tail_body(g_ref, ef_ref, w1e_ref, b1_ref, w2_ref, b2_ref, out_ref):
    x = (g_ref[...]
         + jnp.dot(ef_ref[...], w1e_ref[...],
                   preferred_element_type=jnp.float32)
         + b1_ref[...])
    h = x * jax.nn.sigmoid(x)
    out_ref[...] = (jnp.dot(h, w2_ref[...],
                            preferred_element_type=jnp.float32)
                    + b2_ref[...])


def _mlp_tail(g, edge_feats, w1e, b1, w2, b2):
    nblk = g.shape[0] // _E_BLK
    return pl.pallas_call(
        _tail_body,
        grid=(nblk,),
        in_specs=[
            pl.BlockSpec((_E_BLK, D_HIDDEN), lambda i: (i, 0)),
            pl.BlockSpec((_E_BLK, D_EDGE), lambda i: (i, 0)),
            pl.BlockSpec((D_EDGE, D_HIDDEN), lambda i: (0, 0)),
            pl.BlockSpec((1, D_HIDDEN), lambda i: (0, 0)),
            pl.BlockSpec((D_HIDDEN, D_EDGE), lambda i: (0, 0)),
            pl.BlockSpec((1, D_EDGE), lambda i: (0, 0)),
        ],
        out_specs=pl.BlockSpec((_E_BLK, D_EDGE), lambda i: (i, 0)),
        out_shape=jax.ShapeDtypeStruct((g.shape[0], D_EDGE), jnp.float32),
    )(g, edge_feats, w1e, b1, w2, b2)


def kernel(node_scalars, edge_index, edge_feats, W1, b1, W2, b2):
    src = edge_index[0].astype(jnp.int32)
    dst = edge_index[1].astype(jnp.int32)
    w1_nodes = jnp.stack([W1[:D_SCALAR], W1[D_SCALAR:2 * D_SCALAR]])
    w1e = W1[2 * D_SCALAR:]
    b1r = b1.reshape(1, D_HIDDEN)
    b2r = b2.reshape(1, D_EDGE)
    table = _node_tables(node_scalars, w1_nodes)
    gather = _gather_add_build()
    outs = []
    for k in range(_K):
        lo = k * _E_CHUNK
        g_k = gather(table, lax.slice(src, (lo,), (lo + _E_CHUNK,)),
                     lax.slice(dst, (lo,), (lo + _E_CHUNK,)))
        outs.append(_mlp_tail(
            g_k, lax.slice(edge_feats, (lo, 0), (lo + _E_CHUNK, D_EDGE)),
            w1e, b1r, W2, b2r))
    return jnp.concatenate(outs, axis=0)


# speedup vs baseline: 1.6944x; 1.6944x over previous
import functools

import numpy as np

import jax
import jax.numpy as jnp
from jax import lax
from jax.experimental import pallas as pl
from jax.experimental.pallas import tpu as pltpu
from jax.experimental.pallas import tpu_sc as plsc

N_NODES = 10000
N_EDGES = 320000
D_SCALAR = 128
D_EDGE = 16
D_HIDDEN = 128

_PERM = np.empty(D_HIDDEN, dtype=np.int32)
for _g in range(4):
    for _p in range(32):
        _PERM[32 * _g + _p] = 32 * _g + 16 * (_p % 2) + _p // 2
_PERM_INV = np.argsort(_PERM).astype(np.int32)

_A_BLK = 1000


def _tables_body(ns_ref, w_ref, out_ref):
    out_ref[...] = jnp.dot(ns_ref[...], w_ref[...],
                           preferred_element_type=jnp.float32)


def _node_tables(node_scalars, W1):
    return pl.pallas_call(
        _tables_body,
        grid=(2, N_NODES // _A_BLK),
        in_specs=[
            pl.BlockSpec((_A_BLK, D_SCALAR), lambda t, i: (i, 0)),
            pl.BlockSpec((D_SCALAR, D_HIDDEN), lambda t, i: (t, 0)),
        ],
        out_specs=pl.BlockSpec((_A_BLK, D_HIDDEN),
                               lambda t, i: (t * (N_NODES // _A_BLK) + i, 0)),
        out_shape=jax.ShapeDtypeStruct((2 * N_NODES, D_HIDDEN), jnp.float32),
    )(node_scalars, W1)


_NW = 32
_K = 1
_E_CHUNK = N_EDGES // _K
_EP = _E_CHUNK // _NW
_C = 400
_NCHUNK = _EP // _C


_GATHER_ADD_CACHE = {}


def _gather_add_build(ebase):
    if ebase in _GATHER_ADD_CACHE:
        return _GATHER_ADD_CACHE[ebase]
    mesh = plsc.VectorSubcoreMesh(core_axis_name="c", subcore_axis_name="s")

    @functools.partial(
        pl.kernel,
        out_type=jax.ShapeDtypeStruct((_E_CHUNK, D_HIDDEN), jnp.float32),
        mesh=mesh,
        scratch_types=[
            pltpu.VMEM((_EP,), jnp.int32),
            pltpu.VMEM((_EP,), jnp.int32),
            pltpu.VMEM((_C, D_HIDDEN), jnp.float32),
            pltpu.VMEM((_C, D_HIDDEN), jnp.float32),
            pltpu.SemaphoreType.DMA,
            pltpu.SemaphoreType.DMA,
            pltpu.SemaphoreType.DMA,
            pltpu.SemaphoreType.DMA,
        ],
    )
    def gather_add(table_hbm, src_hbm, dst_hbm, out_hbm,
                   idx_s, idx_d, buf0, buf1, gs0, gs1, ws0, ws1):
        wid = lax.axis_index("s") * 2 + lax.axis_index("c")
        base = pl.multiple_of(wid * _EP, 8)
        bufs = (buf0, buf1)
        gsems = (gs0, gs1)
        wsems = (ws0, ws1)

        pltpu.sync_copy(src_hbm.at[pl.ds(ebase + base, _EP)], idx_s)
        pltpu.sync_copy(dst_hbm.at[pl.ds(ebase + base, _EP)], idx_d)

        def off_row(i, c):
            sl = pl.ds(i * 16, 16)
            idx_d[sl] = idx_d[sl] + N_NODES
            return c

        lax.fori_loop(0, _EP // 16, off_row, 0)

        def g1(ci):
            s = ci % 2
            return pltpu.async_copy(
                table_hbm.at[idx_s.at[pl.ds(ci * _C, _C)]], bufs[s], gsems[s])

        def g2(ci):
            s = ci % 2
            return pltpu.async_copy(
                table_hbm.at[idx_d.at[pl.ds(ci * _C, _C)]], bufs[s], gsems[s],
                add=True)

        def wb(ci):
            s = ci % 2
            return pltpu.async_copy(
                bufs[s], out_hbm.at[pl.ds(base + ci * _C, _C)], wsems[s])

        wbd = [None] * _NCHUNK
        d = g1(0)
        for ci in range(_NCHUNK):
            d.wait()
            dg2 = g2(ci)
            if ci >= 1:
                wbd[ci - 1].wait()
            if ci + 1 < _NCHUNK:
                d = g1(ci + 1)
            dg2.wait()
            wbd[ci] = wb(ci)
        wbd[_NCHUNK - 1].wait()

    _GATHER_ADD_CACHE[ebase] = gather_add
    return gather_add


_E_BLK = 6400


def _tail_body(g_ref, eft_ref, w1e_ref, b1_ref, w2_ref, b2t_ref, out_ref):
    x = (g_ref[...]
         + lax.dot_general(eft_ref[...], w1e_ref[...],
                           (((0,), (0,)), ((), ())),
                           preferred_element_type=jnp.float32)
         + b1_ref[...])
    h = x * jax.nn.sigmoid(x)
    out_ref[...] = (lax.dot_general(w2_ref[...], h,
                                    (((0,), (1,)), ((), ())),
                                    preferred_element_type=jnp.float32)
                    + b2t_ref[...])


def _mlp_tail(g, ef_t, w1e, b1, w2, b2t):
    n = g.shape[0]
    nblk = n // _E_BLK
    return pl.pallas_call(
        _tail_body,
        grid=(nblk,),
        in_specs=[
            pl.BlockSpec((_E_BLK, D_HIDDEN), lambda i: (i, 0)),
            pl.BlockSpec((D_EDGE, _E_BLK), lambda i: (0, i)),
            pl.BlockSpec((D_EDGE, D_HIDDEN), lambda i: (0, 0)),
            pl.BlockSpec((1, D_HIDDEN), lambda i: (0, 0)),
            pl.BlockSpec((D_HIDDEN, D_EDGE), lambda i: (0, 0)),
            pl.BlockSpec((D_EDGE, 1), lambda i: (0, 0)),
        ],
        out_specs=pl.BlockSpec((D_EDGE, _E_BLK), lambda i: (0, i)),
        out_shape=jax.ShapeDtypeStruct((D_EDGE, n), jnp.float32),
    )(g, ef_t, w1e, b1, w2, b2t)


def kernel(node_scalars, edge_index, edge_feats, W1, b1, W2, b2):
    ei = edge_index.astype(jnp.int32)
    src = ei[0]
    dst = ei[1]
    ef_t = edge_feats.T
    w1e = W1[2 * D_SCALAR:]
    b1r = b1.reshape(1, D_HIDDEN)
    b2t = b2.reshape(D_EDGE, 1)
    table = _node_tables(node_scalars, W1)
    outs = []
    for k in range(_K):
        lo = k * _E_CHUNK
        g_k = _gather_add_build(lo)(table, src, dst)
        outs.append(_mlp_tail(
            g_k, lax.slice(ef_t, (0, lo), (D_EDGE, lo + _E_CHUNK)),
            w1e, b1r, W2, b2t))
    if _K == 1:
        return outs[0].T
    return jnp.concatenate(outs, axis=1).T
